# baseline (speedup 1.0000x reference)
import jax
import jax.numpy as jnp
from jax import lax
from jax.experimental import pallas as pl
from jax.experimental.pallas import tpu as pltpu
from jax.experimental.pallas import tpu_sc as plsc

VOCAB = 100000
EMBED_DIM = 300
NUM_CORES = 2
NUM_SUBCORES = 16
NUM_WORKERS = NUM_CORES * NUM_SUBCORES
NBATCH = 4096
SEQ = 50
SEQ_PAD = 56
BATCH_PER_WORKER = NBATCH // NUM_WORKERS
TOTAL_PAD = NBATCH * SEQ_PAD
OUT_W = 384


def _body(table_hbm, tail_hbm, idx_hbm, out_hbm,
          idx_v, rows_a, rows_b, ga, gb, wsem):
    c = lax.axis_index("c")
    s = lax.axis_index("s")
    wid = s * NUM_CORES + c
    pltpu.sync_copy(
        idx_hbm.at[pl.ds(wid * BATCH_PER_WORKER * SEQ_PAD,
                         BATCH_PER_WORKER * SEQ_PAD)], idx_v)
    base = wid * BATCH_PER_WORKER

    def gather(b, rows_v, sem):
        idxs = idx_v.at[pl.ds(b * SEQ_PAD, SEQ_PAD)]
        pltpu.async_copy(table_hbm.at[idxs, pl.ds(0, 128)],
                         rows_v.at[:, pl.ds(0, 128)], sem)
        pltpu.async_copy(table_hbm.at[idxs, pl.ds(128, 128)],
                         rows_v.at[:, pl.ds(128, 128)], sem)
        pltpu.async_copy(tail_hbm.at[idxs],
                         rows_v.at[:, pl.ds(256, 128)], sem)

    def drain3(rows_v, sem):
        for _ in range(3):
            pltpu.make_async_copy(
                table_hbm.at[pl.ds(0, SEQ_PAD), pl.ds(0, 128)],
                rows_v.at[:, pl.ds(0, 128)], sem).wait()

    def write(b, rows_v):
        return pltpu.async_copy(
            rows_v, out_hbm.at[pl.ds((base + b) * SEQ_PAD, SEQ_PAD)], wsem)

    gather(0, rows_a, ga)

    def steppair(i, carry):
        b0 = 2 * i
        gather(b0 + 1, rows_b, gb)
        drain3(rows_a, ga)
        wa = write(b0, rows_a)
        drain3(rows_b, gb)
        wa.wait()

        @pl.when(b0 + 2 < BATCH_PER_WORKER)
        def _():
            gather(b0 + 2, rows_a, ga)

        wb = write(b0 + 1, rows_b)
        wb.wait()
        return carry

    lax.fori_loop(0, BATCH_PER_WORKER // 2, steppair, 0)


RL_B = 8


def _rl_body(in_ref, out_ref):
    out_ref[...] = in_ref[:, :SEQ, :EMBED_DIM]


def _relayout(out_pad3):
    return pl.pallas_call(
        _rl_body,
        out_shape=jax.ShapeDtypeStruct((NBATCH, SEQ, EMBED_DIM), jnp.float32),
        grid=(NBATCH // RL_B,),
        in_specs=[pl.BlockSpec((RL_B, SEQ_PAD, OUT_W), lambda i: (i, 0, 0))],
        out_specs=pl.BlockSpec((RL_B, SEQ, EMBED_DIM), lambda i: (i, 0, 0)),
    )(out_pad3)


@jax.jit
def _run(table, tail, idx_flat):
    mesh = plsc.VectorSubcoreMesh(
        core_axis_name="c", subcore_axis_name="s",
        num_cores=NUM_CORES, num_subcores=NUM_SUBCORES)
    f = pl.kernel(
        _body,
        out_type=jax.ShapeDtypeStruct((TOTAL_PAD, OUT_W), jnp.float32),
        mesh=mesh,
        scratch_types=[
            pltpu.VMEM((BATCH_PER_WORKER * SEQ_PAD,), jnp.int32),
            pltpu.VMEM((SEQ_PAD, OUT_W), jnp.float32),
            pltpu.VMEM((SEQ_PAD, OUT_W), jnp.float32),
            pltpu.SemaphoreType.DMA,
            pltpu.SemaphoreType.DMA,
            pltpu.SemaphoreType.DMA,
        ],
    )
    out_pad = f(table, tail, idx_flat)
    return _relayout(out_pad.reshape(NBATCH, SEQ_PAD, OUT_W))


def kernel(table, pad_indexes):
    idx_pad = jnp.pad(pad_indexes.astype(jnp.int32), ((0, 0), (0, SEQ_PAD - SEQ)))
    idx_flat = idx_pad.reshape(TOTAL_PAD)
    tail = jnp.pad(jax.lax.slice(table, (0, 256), (VOCAB, EMBED_DIM)),
                   ((0, 0), (0, 84)))
    return _run(table, tail, idx_flat)


# speedup vs baseline: 2.9938x; 2.9938x over previous
import jax
import jax.numpy as jnp
from jax import lax
from jax.experimental import pallas as pl
from jax.experimental.pallas import tpu as pltpu
from jax.experimental.pallas import tpu_sc as plsc

VOCAB = 100000
EMBED_DIM = 300
NUM_CORES = 2
NUM_SUBCORES = 16
NUM_WORKERS = NUM_CORES * NUM_SUBCORES
NBATCH = 4096
SEQ = 50
SEQ_PAD = 56
BATCH_PER_WORKER = (NBATCH // 2) // NUM_WORKERS
TOTAL_PAD = NBATCH * SEQ_PAD
OUT_W = 384
PAIR_ROWS = 2 * SEQ_PAD
PAIRS_PER_WORKER = BATCH_PER_WORKER // 2


def _body(table_hbm, tail_hbm, idx_hbm, out_hbm,
          idx_v, rows_a, rows_b, ga, gb, wsem):
    c = lax.axis_index("c")
    s = lax.axis_index("s")
    wid = s * NUM_CORES + c
    pltpu.sync_copy(
        idx_hbm.at[pl.ds(wid * BATCH_PER_WORKER * SEQ_PAD,
                         BATCH_PER_WORKER * SEQ_PAD)], idx_v)
    base = wid * BATCH_PER_WORKER

    def gather(b, rows_v, sem):
        idxs = idx_v.at[pl.ds(b * PAIR_ROWS, PAIR_ROWS)]
        pltpu.async_copy(table_hbm.at[idxs, pl.ds(0, 128)],
                         rows_v.at[:, pl.ds(0, 128)], sem)
        pltpu.async_copy(table_hbm.at[idxs, pl.ds(128, 128)],
                         rows_v.at[:, pl.ds(128, 128)], sem)
        pltpu.async_copy(tail_hbm.at[idxs],
                         rows_v.at[:, pl.ds(256, 128)], sem)

    def drain3(rows_v, sem):
        for _ in range(3):
            pltpu.make_async_copy(
                table_hbm.at[pl.ds(0, PAIR_ROWS), pl.ds(0, 128)],
                rows_v.at[:, pl.ds(0, 128)], sem).wait()

    def write(b, rows_v):
        return pltpu.async_copy(
            rows_v,
            out_hbm.at[pl.ds(base * SEQ_PAD + b * PAIR_ROWS, PAIR_ROWS)],
            wsem)

    gather(0, rows_a, ga)

    def steppair(i, carry):
        b0 = 2 * i
        gather(b0 + 1, rows_b, gb)
        drain3(rows_a, ga)
        wa = write(b0, rows_a)
        drain3(rows_b, gb)
        wa.wait()

        @pl.when(b0 + 2 < PAIRS_PER_WORKER)
        def _():
            gather(b0 + 2, rows_a, ga)

        wb = write(b0 + 1, rows_b)
        wb.wait()
        return carry

    lax.fori_loop(0, PAIRS_PER_WORKER // 2, steppair, 0)


RL_B = 8


def _rl_body(in_ref, out_ref):
    out_ref[...] = in_ref[:, :SEQ, :EMBED_DIM]


def _relayout(out_pad3):
    return pl.pallas_call(
        _rl_body,
        out_shape=jax.ShapeDtypeStruct((NBATCH, SEQ, EMBED_DIM), jnp.float32),
        grid=(NBATCH // RL_B,),
        in_specs=[pl.BlockSpec((RL_B, SEQ_PAD, OUT_W), lambda i: (i, 0, 0))],
        out_specs=pl.BlockSpec((RL_B, SEQ, EMBED_DIM), lambda i: (i, 0, 0)),
    )(out_pad3)


NHALF = NBATCH // 2
HALF_PAD_ROWS = NHALF * SEQ_PAD


@jax.jit
def _run(table, tail, idx_flat):
    mesh = plsc.VectorSubcoreMesh(
        core_axis_name="c", subcore_axis_name="s",
        num_cores=NUM_CORES, num_subcores=NUM_SUBCORES)
    f = pl.kernel(
        _body,
        out_type=jax.ShapeDtypeStruct((HALF_PAD_ROWS, OUT_W), jnp.float32),
        mesh=mesh,
        scratch_types=[
            pltpu.VMEM((BATCH_PER_WORKER * SEQ_PAD, ), jnp.int32),
            pltpu.VMEM((PAIR_ROWS, OUT_W), jnp.float32),
            pltpu.VMEM((PAIR_ROWS, OUT_W), jnp.float32),
            pltpu.SemaphoreType.DMA,
            pltpu.SemaphoreType.DMA,
            pltpu.SemaphoreType.DMA,
        ],
    )
    out_a = f(table, tail, idx_flat[:NHALF * SEQ_PAD])
    out_b = f(table, tail, idx_flat[NHALF * SEQ_PAD:])
    va = out_a.reshape(NHALF, SEQ_PAD, OUT_W)[:, :SEQ, :EMBED_DIM]
    vb = out_b.reshape(NHALF, SEQ_PAD, OUT_W)[:, :SEQ, :EMBED_DIM]
    return jnp.concatenate([va, vb], axis=0)


def kernel(table, pad_indexes):
    idx32 = pad_indexes.astype(jnp.int32)
    idx_pad = jnp.concatenate([idx32, idx32[:, :SEQ_PAD - SEQ]], axis=1)
    idx_flat = idx_pad.reshape(TOTAL_PAD)
    tail = jnp.pad(jax.lax.slice(table, (0, 256), (VOCAB, EMBED_DIM)),
                   ((0, 0), (0, 84)))
    return _run(table, tail, idx_flat)


# speedup vs baseline: 3.7378x; 1.2485x over previous
import jax
import jax.numpy as jnp
from jax import lax
from jax.experimental import pallas as pl
from jax.experimental.pallas import tpu as pltpu
from jax.experimental.pallas import tpu_sc as plsc

VOCAB = 100000
EMBED_DIM = 300
NUM_CORES = 2
NUM_SUBCORES = 16
NUM_WORKERS = NUM_CORES * NUM_SUBCORES
NBATCH = 4096
SEQ = 50
SEQ_PAD = 56
BATCH_PER_WORKER = NBATCH // NUM_WORKERS
TOTAL_PAD = NBATCH * SEQ_PAD
OUT_W = 384
PAIR_ROWS = 2 * SEQ_PAD
PAIRS_PER_WORKER = BATCH_PER_WORKER // 2


def _body(table_hbm, tail_hbm, idx_hbm, out_hbm,
          idx_v, rows_a, rows_b, ga, gb, wsem):
    c = lax.axis_index("c")
    s = lax.axis_index("s")
    wid = s * NUM_CORES + c
    pltpu.sync_copy(
        idx_hbm.at[pl.ds(wid * BATCH_PER_WORKER * SEQ_PAD,
                         BATCH_PER_WORKER * SEQ_PAD)], idx_v)
    base = wid * BATCH_PER_WORKER

    def gather(b, rows_v, sem):
        idxs = idx_v.at[pl.ds(b * PAIR_ROWS, PAIR_ROWS)]
        pltpu.async_copy(table_hbm.at[idxs, pl.ds(0, 128)],
                         rows_v.at[:, pl.ds(0, 128)], sem)
        pltpu.async_copy(table_hbm.at[idxs, pl.ds(128, 128)],
                         rows_v.at[:, pl.ds(128, 128)], sem)
        pltpu.async_copy(tail_hbm.at[idxs],
                         rows_v.at[:, pl.ds(256, 128)], sem)

    def drain3(rows_v, sem):
        for _ in range(3):
            pltpu.make_async_copy(
                table_hbm.at[pl.ds(0, PAIR_ROWS), pl.ds(0, 128)],
                rows_v.at[:, pl.ds(0, 128)], sem).wait()

    def write(b, rows_v):
        return pltpu.async_copy(
            rows_v,
            out_hbm.at[pl.ds(base * SEQ_PAD + b * PAIR_ROWS, PAIR_ROWS)],
            wsem)

    gather(0, rows_a, ga)

    def steppair(i, carry):
        b0 = 2 * i
        gather(b0 + 1, rows_b, gb)
        drain3(rows_a, ga)
        wa = write(b0, rows_a)
        drain3(rows_b, gb)
        wa.wait()

        @pl.when(b0 + 2 < PAIRS_PER_WORKER)
        def _():
            gather(b0 + 2, rows_a, ga)

        wb = write(b0 + 1, rows_b)
        wb.wait()
        return carry

    lax.fori_loop(0, PAIRS_PER_WORKER // 2, steppair, 0)


@jax.jit
def _run(table, tail, idx_flat):
    mesh = plsc.VectorSubcoreMesh(
        core_axis_name="c", subcore_axis_name="s",
        num_cores=NUM_CORES, num_subcores=NUM_SUBCORES)
    f = pl.kernel(
        _body,
        out_type=jax.ShapeDtypeStruct((TOTAL_PAD, OUT_W), jnp.float32),
        mesh=mesh,
        scratch_types=[
            pltpu.VMEM((BATCH_PER_WORKER * SEQ_PAD,), jnp.int32),
            pltpu.VMEM((PAIR_ROWS, OUT_W), jnp.float32),
            pltpu.VMEM((PAIR_ROWS, OUT_W), jnp.float32),
            pltpu.SemaphoreType.DMA,
            pltpu.SemaphoreType.DMA,
            pltpu.SemaphoreType.DMA,
        ],
    )
    out_pad = f(table, tail, idx_flat)
    return out_pad.reshape(NBATCH, SEQ_PAD, OUT_W)[:, :SEQ, :EMBED_DIM]


def kernel(table, pad_indexes):
    idx32 = pad_indexes.astype(jnp.int32)
    idx_pad = jnp.concatenate([idx32, idx32[:, :SEQ_PAD - SEQ]], axis=1)
    idx_flat = idx_pad.reshape(TOTAL_PAD)
    tail = jnp.pad(jax.lax.slice(table, (0, 256), (VOCAB, EMBED_DIM)),
                   ((0, 0), (0, 84)))
    return _run(table, tail, idx_flat)
